# baseline (speedup 1.0000x reference)
import functools

import jax
import jax.numpy as jnp
from jax import lax
from jax.experimental import pallas as pl
from jax.experimental.pallas import tpu as pltpu
from jax.experimental.pallas import tpu_sc as plsc

D = 64
DP = 128
VOC = 1000000
NC = 2
NS = 16
NW = NC * NS
CI = 1024
CG = 256
NSUB = CI // CG


def _make_sc_gather(B: int):
    b_per_w = B // NW
    nchunk = b_per_w // CI
    assert B % NW == 0 and b_per_w % CI == 0 and nchunk >= 3 and nchunk % 2 == 1
    mesh = plsc.VectorSubcoreMesh(core_axis_name="c", subcore_axis_name="s")

    @functools.partial(
        pl.kernel,
        out_type=jax.ShapeDtypeStruct((B, DP), jnp.float32),
        mesh=mesh,
        scratch_types=(
            [pltpu.VMEM((CI,), jnp.int32) for _ in range(2)]
            + [pltpu.VMEM((CG, DP), jnp.float32) for _ in range(2)]
            + [pltpu.SemaphoreType.DMA] * 6
        ),
        compiler_params=pltpu.CompilerParams(use_tc_tiling_on_sc=True),
    )
    def sc_gather(table_hbm, tok_hbm, out_hbm, idx0, idx1, rows0, rows1,
                  si0, si1, sg0, sg1, ss0, ss1):
        idxs = (idx0, idx1)
        rows = (rows0, rows1)
        si = (si0, si1)
        sg = (sg0, sg1)
        ss = (ss0, ss1)
        wid = lax.axis_index("s") * NC + lax.axis_index("c")
        wbase = pl.multiple_of(wid * b_per_w, CI)

        def idx_desc(c, p):
            base = pl.multiple_of(wbase + c * CI, CI)
            return pltpu.make_async_copy(
                tok_hbm.at[pl.ds(base, CI)], idxs[p], si[p])

        def gather_desc(k, p):
            return pltpu.make_async_copy(
                table_hbm.at[idxs[p].at[pl.ds(k * CG, CG)]],
                rows[k % 2], sg[k % 2])

        def scatter_desc(c, k):
            base = pl.multiple_of(wbase + c * CI + k * CG, 8)
            return pltpu.make_async_copy(
                rows[k % 2], out_hbm.at[pl.ds(base, CG)], ss[k % 2])

        def chunk_body(c, p, first):
            if first:
                idx_desc(c, p).wait()
                gather_desc(0, p).start()
                idx_desc(c + 1, 1 - p).start()
                for k in (1, 2, 3):
                    gather_desc(k - 1, p).wait()
                    scatter_desc(c, k - 1).start()
                    if k >= 2:
                        scatter_desc(c, k - 2).wait()
                    gather_desc(k, p).start()
            else:
                gather_desc(3, 1 - p).wait()
                scatter_desc(c - 1, 3).start()
                c_next = jnp.minimum(c + 1, nchunk - 1)
                idx_desc(c_next, 1 - p).start()
                idx_desc(c, p).wait()
                scatter_desc(c - 1, 2).wait()
                gather_desc(0, p).start()
                for k in (1, 2, 3):
                    gather_desc(k - 1, p).wait()
                    scatter_desc(c, k - 1).start()
                    cw, kw = (c, k - 2) if k >= 2 else (c - 1, 3)
                    scatter_desc(cw, kw).wait()
                    gather_desc(k, p).start()

        idx_desc(0, 0).start()
        chunk_body(0, 0, True)

        @pl.loop(1, nchunk, step=2)
        def _(c0):
            chunk_body(c0, 1, False)
            chunk_body(c0 + 1, 0, False)

        gather_desc(3, (nchunk - 1) % 2).wait()
        scatter_desc(nchunk - 1, 3).start()
        scatter_desc(nchunk - 1, 2).wait()
        scatter_desc(nchunk - 1, 3).wait()
        idx_desc(nchunk - 1, nchunk % 2).wait()

    return sc_gather


_PAD_BLK = 1000


def _pad_table(w):
    V = w.shape[0]

    def body(in_ref, out_ref):
        x = in_ref[...]
        out_ref[...] = jnp.concatenate([x, jnp.zeros_like(x)], axis=1)

    return pl.pallas_call(
        body,
        grid=(V // _PAD_BLK,),
        in_specs=[pl.BlockSpec((_PAD_BLK, D), lambda i: (i, 0))],
        out_specs=pl.BlockSpec((_PAD_BLK, DP), lambda i: (i, 0)),
        out_shape=jax.ShapeDtypeStruct((V, DP), jnp.float32),
    )(w)


def kernel(tokens, embed_weight):
    S, T = tokens.shape
    B = S * T
    table128 = _pad_table(embed_weight)
    flat = tokens.reshape(B)
    out = _make_sc_gather(B)(table128, flat)
    return out[:, :D].reshape(S, T, D)


# speedup vs baseline: 1.3111x; 1.3111x over previous
import functools

import jax
import jax.numpy as jnp
from jax import lax
from jax.experimental import pallas as pl
from jax.experimental.pallas import tpu as pltpu
from jax.experimental.pallas import tpu_sc as plsc

D = 64
DP = 128
NC = 2
NS = 16
NW = NC * NS
CI = 1024
CG = 128
NSUB = CI // CG


def _make_sc_gather(B: int):
    b_per_w = B // NW
    nchunk = b_per_w // CI
    assert B % NW == 0 and b_per_w % CI == 0 and nchunk >= 3 and nchunk % 2 == 1
    mesh = plsc.VectorSubcoreMesh(core_axis_name="c", subcore_axis_name="s")

    @functools.partial(
        pl.kernel,
        out_type=jax.ShapeDtypeStruct((B, D), jnp.float32),
        mesh=mesh,
        scratch_types=(
            [pltpu.VMEM((CI,), jnp.int32) for _ in range(2)]
            + [pltpu.VMEM((CG, DP), jnp.float32) for _ in range(2)]
            + [pltpu.VMEM((CG, D), jnp.float32) for _ in range(2)]
            + [pltpu.SemaphoreType.DMA] * 6
        ),
        compiler_params=pltpu.CompilerParams(use_tc_tiling_on_sc=True),
    )
    def sc_gather(table_hbm, tok_hbm, out_hbm, idx0, idx1, rows0, rows1,
                  cmp0, cmp1, si0, si1, sg0, sg1, ss0, ss1):
        idxs = (idx0, idx1)
        rows = (rows0, rows1)
        cmps = (cmp0, cmp1)
        si = (si0, si1)
        sg = (sg0, sg1)
        ss = (ss0, ss1)
        wid = lax.axis_index("s") * NC + lax.axis_index("c")
        wbase = pl.multiple_of(wid * b_per_w, CI)

        def idx_desc(c, p):
            base = pl.multiple_of(wbase + c * CI, CI)
            return pltpu.make_async_copy(
                tok_hbm.at[pl.ds(base, CI)], idxs[p], si[p])

        def gather_desc(k, p):
            return pltpu.make_async_copy(
                table_hbm.at[idxs[p].at[pl.ds(k * CG, CG)]],
                rows[k % 2], sg[k % 2])

        def compact(k):
            src = rows[k % 2]
            dst = cmps[k % 2]

            @pl.loop(0, CG)
            def _(r):
                for j in range(D // 16):
                    dst[r, pl.ds(16 * j, 16)] = src[r, pl.ds(16 * j, 16)]

        def scatter_desc(c, k):
            base = pl.multiple_of(wbase + c * CI + k * CG, 8)
            return pltpu.make_async_copy(
                cmps[k % 2], out_hbm.at[pl.ds(base, CG)], ss[k % 2])

        def chunk_body(c, p, first):
            if first:
                idx_desc(c, p).wait()
                gather_desc(0, p).start()
                idx_desc(c + 1, 1 - p).start()
                for k in range(1, NSUB):
                    gather_desc(k - 1, p).wait()
                    compact(k - 1)
                    scatter_desc(c, k - 1).start()
                    if k >= 2:
                        scatter_desc(c, k - 2).wait()
                    gather_desc(k, p).start()
            else:
                gather_desc(NSUB - 1, 1 - p).wait()
                compact(NSUB - 1)
                scatter_desc(c - 1, NSUB - 1).start()
                c_next = jnp.minimum(c + 1, nchunk - 1)
                idx_desc(c_next, 1 - p).start()
                idx_desc(c, p).wait()
                scatter_desc(c - 1, NSUB - 2).wait()
                gather_desc(0, p).start()
                for k in range(1, NSUB):
                    gather_desc(k - 1, p).wait()
                    compact(k - 1)
                    scatter_desc(c, k - 1).start()
                    cw, kw = (c, k - 2) if k >= 2 else (c - 1, NSUB - 1)
                    scatter_desc(cw, kw).wait()
                    gather_desc(k, p).start()

        idx_desc(0, 0).start()
        chunk_body(0, 0, True)

        @pl.loop(1, nchunk, step=2)
        def _(c0):
            chunk_body(c0, 1, False)
            chunk_body(c0 + 1, 0, False)

        gather_desc(NSUB - 1, (nchunk - 1) % 2).wait()
        compact(NSUB - 1)
        scatter_desc(nchunk - 1, NSUB - 1).start()
        scatter_desc(nchunk - 1, NSUB - 2).wait()
        scatter_desc(nchunk - 1, NSUB - 1).wait()
        idx_desc(nchunk - 1, nchunk % 2).wait()

    return sc_gather


def kernel(tokens, embed_weight):
    S, T = tokens.shape
    B = S * T
    table128 = jnp.pad(embed_weight, ((0, 0), (0, DP - D)))
    flat = tokens.reshape(B)
    out = _make_sc_gather(B)(table128, flat)
    return out.reshape(S, T, D)


# speedup vs baseline: 1.3200x; 1.0068x over previous
import functools

import jax
import jax.numpy as jnp
from jax import lax
from jax.experimental import pallas as pl
from jax.experimental.pallas import tpu as pltpu
from jax.experimental.pallas import tpu_sc as plsc

D = 64
DP = 128
NC = 2
NS = 16
NW = NC * NS
CI = 1024
CG = 256
NSUB = CI // CG


def _make_sc_gather(B: int):
    b_per_w = B // NW
    nchunk = b_per_w // CI
    assert B % NW == 0 and b_per_w % CI == 0 and nchunk >= 3 and nchunk % 2 == 1
    mesh = plsc.VectorSubcoreMesh(core_axis_name="c", subcore_axis_name="s")

    @functools.partial(
        pl.kernel,
        out_type=jax.ShapeDtypeStruct((B, DP), jnp.float32),
        mesh=mesh,
        scratch_types=(
            [pltpu.VMEM((CI,), jnp.int32) for _ in range(2)]
            + [pltpu.VMEM((CG, DP), jnp.float32) for _ in range(2)]
            + [pltpu.SemaphoreType.DMA] * 6
        ),
        compiler_params=pltpu.CompilerParams(use_tc_tiling_on_sc=True),
    )
    def sc_gather(table_hbm, tok_hbm, out_hbm, idx0, idx1, rows0, rows1,
                  si0, si1, sg0, sg1, ss0, ss1):
        idxs = (idx0, idx1)
        rows = (rows0, rows1)
        si = (si0, si1)
        sg = (sg0, sg1)
        ss = (ss0, ss1)
        wid = lax.axis_index("s") * NC + lax.axis_index("c")
        wbase = pl.multiple_of(wid * b_per_w, CI)

        def idx_desc(c, p):
            base = pl.multiple_of(wbase + c * CI, CI)
            return pltpu.make_async_copy(
                tok_hbm.at[pl.ds(base, CI)], idxs[p], si[p])

        def gather_desc(k, p):
            return pltpu.make_async_copy(
                table_hbm.at[idxs[p].at[pl.ds(k * CG, CG)]],
                rows[k % 2], sg[k % 2])

        def scatter_desc(c, k):
            base = pl.multiple_of(wbase + c * CI + k * CG, 8)
            return pltpu.make_async_copy(
                rows[k % 2], out_hbm.at[pl.ds(base, CG)], ss[k % 2])

        def chunk_body(c, p, first):
            if first:
                idx_desc(c, p).wait()
                gather_desc(0, p).start()
                idx_desc(c + 1, 1 - p).start()
                for k in (1, 2, 3):
                    gather_desc(k - 1, p).wait()
                    scatter_desc(c, k - 1).start()
                    if k >= 2:
                        scatter_desc(c, k - 2).wait()
                    gather_desc(k, p).start()
            else:
                gather_desc(3, 1 - p).wait()
                scatter_desc(c - 1, 3).start()
                c_next = jnp.minimum(c + 1, nchunk - 1)
                idx_desc(c_next, 1 - p).start()
                idx_desc(c, p).wait()
                scatter_desc(c - 1, 2).wait()
                gather_desc(0, p).start()
                for k in (1, 2, 3):
                    gather_desc(k - 1, p).wait()
                    scatter_desc(c, k - 1).start()
                    cw, kw = (c, k - 2) if k >= 2 else (c - 1, 3)
                    scatter_desc(cw, kw).wait()
                    gather_desc(k, p).start()

        idx_desc(0, 0).start()
        chunk_body(0, 0, True)

        @pl.loop(1, nchunk, step=2)
        def _(c0):
            chunk_body(c0, 1, False)
            chunk_body(c0 + 1, 0, False)

        gather_desc(3, (nchunk - 1) % 2).wait()
        scatter_desc(nchunk - 1, 3).start()
        scatter_desc(nchunk - 1, 2).wait()
        scatter_desc(nchunk - 1, 3).wait()
        idx_desc(nchunk - 1, nchunk % 2).wait()

    return sc_gather


_PAD_BLK = 5000


def _pad_table(w):
    V = w.shape[0]

    def body(in_ref, out_ref):
        out_ref[:, 0:D] = in_ref[...]
        out_ref[:, D:DP] = jnp.zeros((_PAD_BLK, D), jnp.float32)

    return pl.pallas_call(
        body,
        grid=(V // _PAD_BLK,),
        in_specs=[pl.BlockSpec((_PAD_BLK, D), lambda i: (i, 0))],
        out_specs=pl.BlockSpec((_PAD_BLK, DP), lambda i: (i, 0)),
        out_shape=jax.ShapeDtypeStruct((V, DP), jnp.float32),
    )(w)


def kernel(tokens, embed_weight):
    S, T = tokens.shape
    B = S * T
    table128 = _pad_table(embed_weight)
    flat = tokens.reshape(B)
    out = _make_sc_gather(B)(table128, flat)
    return out[:, :D].reshape(S, T, D)


# speedup vs baseline: 1.5137x; 1.1467x over previous
import functools

import jax
import jax.numpy as jnp
from jax import lax
from jax.experimental import pallas as pl
from jax.experimental.pallas import tpu as pltpu
from jax.experimental.pallas import tpu_sc as plsc

D = 64
DP = 128
NC = 2
NS = 16
NW = NC * NS
CI = 1024
CG = 256
NSUB = CI // CG


def _make_sc_gather(B: int):
    b_per_w = B // NW
    nchunk = b_per_w // CI
    assert B % NW == 0 and b_per_w % CI == 0 and nchunk >= 3 and nchunk % 2 == 1
    mesh = plsc.VectorSubcoreMesh(core_axis_name="c", subcore_axis_name="s")

    @functools.partial(
        pl.kernel,
        out_type=jax.ShapeDtypeStruct((B, DP), jnp.float32),
        mesh=mesh,
        scratch_types=(
            [pltpu.VMEM((CI,), jnp.int32) for _ in range(2)]
            + [pltpu.VMEM((CG, DP), jnp.float32) for _ in range(2)]
            + [pltpu.SemaphoreType.DMA] * 6
        ),
        compiler_params=pltpu.CompilerParams(use_tc_tiling_on_sc=True),
    )
    def sc_gather(table_hbm, tok_hbm, out_hbm, idx0, idx1, rows0, rows1,
                  si0, si1, sg0, sg1, ss0, ss1):
        idxs = (idx0, idx1)
        rows = (rows0, rows1)
        si = (si0, si1)
        sg = (sg0, sg1)
        ss = (ss0, ss1)
        wid = lax.axis_index("s") * NC + lax.axis_index("c")
        wbase = pl.multiple_of(wid * b_per_w, CI)

        def idx_desc(c, p):
            base = pl.multiple_of(wbase + c * CI, CI)
            return pltpu.make_async_copy(
                tok_hbm.at[pl.ds(base, CI)], idxs[p], si[p])

        def gather_desc(k, p):
            return pltpu.make_async_copy(
                table_hbm.at[idxs[p].at[pl.ds(k * CG, CG)]],
                rows[k % 2], sg[k % 2])

        def scatter_desc(c, k):
            base = pl.multiple_of(wbase + c * CI + k * CG, 8)
            return pltpu.make_async_copy(
                rows[k % 2], out_hbm.at[pl.ds(base, CG)], ss[k % 2])

        def chunk_body(c, p, first):
            if first:
                idx_desc(c, p).wait()
                gather_desc(0, p).start()
                idx_desc(c + 1, 1 - p).start()
                for k in (1, 2, 3):
                    gather_desc(k - 1, p).wait()
                    scatter_desc(c, k - 1).start()
                    if k >= 2:
                        scatter_desc(c, k - 2).wait()
                    gather_desc(k, p).start()
            else:
                gather_desc(3, 1 - p).wait()
                scatter_desc(c - 1, 3).start()
                c_next = jnp.minimum(c + 1, nchunk - 1)
                idx_desc(c_next, 1 - p).start()
                idx_desc(c, p).wait()
                scatter_desc(c - 1, 2).wait()
                gather_desc(0, p).start()
                for k in (1, 2, 3):
                    gather_desc(k - 1, p).wait()
                    scatter_desc(c, k - 1).start()
                    cw, kw = (c, k - 2) if k >= 2 else (c - 1, 3)
                    scatter_desc(cw, kw).wait()
                    gather_desc(k, p).start()

        idx_desc(0, 0).start()
        chunk_body(0, 0, True)

        @pl.loop(1, nchunk, step=2)
        def _(c0):
            chunk_body(c0, 1, False)
            chunk_body(c0 + 1, 0, False)

        gather_desc(3, (nchunk - 1) % 2).wait()
        scatter_desc(nchunk - 1, 3).start()
        scatter_desc(nchunk - 1, 2).wait()
        scatter_desc(nchunk - 1, 3).wait()
        idx_desc(nchunk - 1, nchunk % 2).wait()

    return sc_gather


def kernel(tokens, embed_weight):
    S, T = tokens.shape
    B = S * T
    table128 = jnp.pad(embed_weight, ((0, 0), (0, DP - D)))
    flat = tokens.reshape(B)
    out = _make_sc_gather(B)(table128, flat)
    return out[:, :D].reshape(S, T, D)


# speedup vs baseline: 1.5166x; 1.0019x over previous
import functools

import jax
import jax.numpy as jnp
from jax import lax
from jax.experimental import pallas as pl
from jax.experimental.pallas import tpu as pltpu
from jax.experimental.pallas import tpu_sc as plsc

D = 64
DP = 128
NC = 2
NS = 16
NW = NC * NS
CI = 1024
CG = 128
NSUB = CI // CG
NB = 4


def _make_sc_gather(B: int):
    b_per_w = B // NW
    nchunk = b_per_w // CI
    assert B % NW == 0 and b_per_w % CI == 0 and nchunk >= 3 and nchunk % 2 == 1
    mesh = plsc.VectorSubcoreMesh(core_axis_name="c", subcore_axis_name="s")

    @functools.partial(
        pl.kernel,
        out_type=jax.ShapeDtypeStruct((B, DP), jnp.float32),
        mesh=mesh,
        scratch_types=(
            [pltpu.VMEM((CI,), jnp.int32) for _ in range(2)]
            + [pltpu.VMEM((CG, DP), jnp.float32) for _ in range(NB)]
            + [pltpu.SemaphoreType.DMA] * (2 + 2 * NB)
        ),
        compiler_params=pltpu.CompilerParams(use_tc_tiling_on_sc=True),
    )
    def sc_gather(table_hbm, tok_hbm, out_hbm, idx0, idx1, *rest):
        idxs = (idx0, idx1)
        rows = rest[0:NB]
        si = rest[NB:NB + 2]
        sg = rest[NB + 2:2 * NB + 2]
        ss = rest[2 * NB + 2:3 * NB + 2]
        wid = lax.axis_index("s") * NC + lax.axis_index("c")
        wbase = pl.multiple_of(wid * b_per_w, CI)

        def idx_desc(c, p):
            base = pl.multiple_of(wbase + c * CI, CI)
            return pltpu.make_async_copy(
                tok_hbm.at[pl.ds(base, CI)], idxs[p], si[p])

        def gather_desc(k, p):
            return pltpu.make_async_copy(
                table_hbm.at[idxs[p].at[pl.ds(k * CG, CG)]],
                rows[k % NB], sg[k % NB])

        def scatter_desc(c, k):
            base = pl.multiple_of(wbase + c * CI + k * CG, 8)
            return pltpu.make_async_copy(
                rows[k % NB], out_hbm.at[pl.ds(base, CG)], ss[k % NB])

        def chunk_body(c, p, first):
            for k in range(NSUB):
                if first and k < NB:
                    pass
                else:
                    scatter_desc(*((c, k - NB) if k >= NB else
                                   (c - 1, k - NB + NSUB))).wait()
                if k == 0:
                    idx_desc(c, p).wait()
                gather_desc(k, p).start()
                if first and k < 2:
                    continue
                if k >= 2:
                    gather_desc(k - 2, p).wait()
                    scatter_desc(c, k - 2).start()
                else:
                    gather_desc(k - 2 + NSUB, 1 - p).wait()
                    scatter_desc(c - 1, k - 2 + NSUB).start()
                if k == 1:
                    c_next = jnp.minimum(c + 1, nchunk - 1)
                    idx_desc(c_next, 1 - p).start()

        idx_desc(0, 0).start()
        idx_desc(1, 1).start()
        chunk_body(0, 0, True)

        @pl.loop(1, nchunk, step=2)
        def _(c0):
            chunk_body(c0, 1, False)
            chunk_body(c0 + 1, 0, False)

        cl = nchunk - 1
        pl_ = (nchunk - 1) % 2
        gather_desc(NSUB - 2, pl_).wait()
        scatter_desc(cl, NSUB - 2).start()
        gather_desc(NSUB - 1, pl_).wait()
        scatter_desc(cl, NSUB - 1).start()
        for k in range(NSUB - NB, NSUB):
            scatter_desc(cl, k).wait()
        idx_desc(cl, nchunk % 2).wait()

    return sc_gather


def kernel(tokens, embed_weight):
    S, T = tokens.shape
    B = S * T
    table128 = jnp.pad(embed_weight, ((0, 0), (0, DP - D)))
    flat = tokens.reshape(B)
    out = _make_sc_gather(B)(table128, flat)
    return out[:, :D].reshape(S, T, D)
